# baseline (speedup 1.0000x reference)
import functools
import jax
import jax.numpy as jnp
from jax import lax
from jax.experimental import pallas as pl
from jax.experimental.pallas import tpu as pltpu, tpu_sc as plsc

NUM_PATCHES = 576
LATENT_DIM = 768
BATCH = 64

NC = 2
NS = 16
NW = NC * NS
PP = NUM_PATCHES // NW
CH = PP * LATENT_DIM
ROW = NUM_PATCHES * LATENT_DIM

_mesh = plsc.VectorSubcoreMesh(
    core_axis_name="c", subcore_axis_name="s", num_cores=NC, num_subcores=NS)


@functools.partial(
    pl.kernel,
    out_type=jax.ShapeDtypeStruct((BATCH * ROW,), jnp.float32),
    mesh=_mesh,
    scratch_types=[
        pltpu.VMEM((CH,), jnp.float32),
        pltpu.VMEM((2, CH), jnp.float32),
        pltpu.VMEM((2, CH), jnp.float32),
        pltpu.SemaphoreType.DMA((2,)),
        pltpu.SemaphoreType.DMA((2,)),
    ],
    compiler_params=pltpu.CompilerParams(use_tc_tiling_on_sc=False),
)
def _sc_add(x_hbm, pos_hbm, out_hbm, pbuf, xbuf, obuf, in_sem, out_sem):
    wid = lax.axis_index("s") * NC + lax.axis_index("c")
    e0 = wid * CH
    pltpu.sync_copy(pos_hbm.at[pl.ds(e0, CH)], pbuf)

    def in_copy(b):
        s = b % 2
        return pltpu.make_async_copy(
            x_hbm.at[pl.ds(b * ROW + e0, CH)], xbuf.at[s], in_sem.at[s])

    def out_copy(b):
        s = b % 2
        return pltpu.make_async_copy(
            obuf.at[s], out_hbm.at[pl.ds(b * ROW + e0, CH)], out_sem.at[s])

    in_copy(0).start()
    in_copy(1).start()

    for b in range(BATCH):
        s = b % 2
        in_copy(b).wait()
        if b >= 2:
            out_copy(b - 2).wait()

        @plsc.parallel_loop(0, CH, step=16, unroll=8)
        def _chunks(i):
            obuf[s, pl.ds(i, 16)] = xbuf[s, pl.ds(i, 16)] + pbuf[pl.ds(i, 16)]

        out_copy(b).start()
        if b + 2 < BATCH:
            in_copy(b + 2).start()

    out_copy(BATCH - 2).wait()
    out_copy(BATCH - 1).wait()


def kernel(x, pos_table):
    out = _sc_add(x.reshape(-1), pos_table.reshape(-1))
    return out.reshape(BATCH, NUM_PATCHES, LATENT_DIM)


# speedup vs baseline: 4.8089x; 4.8089x over previous
import jax
import jax.numpy as jnp
from jax.experimental import pallas as pl

NUM_PATCHES = 576
LATENT_DIM = 768
BATCH = 64

BB = 8


def _add_kernel(x_ref, pos_ref, out_ref):
    out_ref[...] = x_ref[...] + pos_ref[...]


def kernel(x, pos_table):
    return pl.pallas_call(
        _add_kernel,
        grid=(BATCH // BB,),
        in_specs=[
            pl.BlockSpec((BB, NUM_PATCHES, LATENT_DIM), lambda b: (b, 0, 0)),
            pl.BlockSpec((NUM_PATCHES, LATENT_DIM), lambda b: (0, 0)),
        ],
        out_specs=pl.BlockSpec((BB, NUM_PATCHES, LATENT_DIM), lambda b: (b, 0, 0)),
        out_shape=jax.ShapeDtypeStruct((BATCH, NUM_PATCHES, LATENT_DIM), x.dtype),
    )(x, pos_table)
